# baseline (speedup 1.0000x reference)
import functools

import jax
import jax.numpy as jnp
from jax import lax
from jax.experimental import pallas as pl
from jax.experimental.pallas import tpu as pltpu
from jax.experimental.pallas import tpu_sc as plsc

NC = 2
NS = 16
NW = NC * NS
B = 16384
D = 64
B_PER_W = B // NW
VS = 244 * 128
CW = 512
NWIN = VS // CW
LO30 = NW * VS
LO31 = LO30 + CW
NSTAGE = 128
OUT128_ROWS = B + NSTAGE

_mesh = plsc.VectorSubcoreMesh(core_axis_name="c", subcore_axis_name="s")


@functools.partial(
    pl.kernel,
    mesh=_mesh,
    out_type=jax.ShapeDtypeStruct((OUT128_ROWS, 128), jnp.float32),
    compiler_params=pltpu.CompilerParams(needs_layout_passes=False),
    scratch_types=[
        pltpu.VMEM((2048,), jnp.int32),
        pltpu.VMEM((B,), jnp.int32),
        pltpu.VMEM((D, CW), jnp.float32),
        pltpu.VMEM((D, CW), jnp.float32),
        pltpu.VMEM((NSTAGE, 128), jnp.float32),
        pltpu.VMEM((1, 128), jnp.int32),
        pltpu.VMEM((D, 128), jnp.float32),
        pltpu.SemaphoreType.DMA,
        pltpu.SemaphoreType.DMA,
    ],
)
def _scan_extract(idx_hbm, tT_hbm, tail_hbm, out128_hbm,
                  ids_v, mrp_v, buf0_v, buf1_v, stage_v, posw_v, tail_v,
                  semw, sems):
    wid = lax.axis_index("s") * NC + lax.axis_index("c")
    v0 = wid * VS
    lanes = lax.iota(jnp.int32, 16)
    is30 = wid == 30
    is31 = wid == 31
    lo2 = jnp.where(is30, LO30, jnp.where(is31, LO31, jnp.int32(1 << 30)))
    hi2 = jnp.where(is30, LO30 + CW, jnp.where(is31, jnp.int32(1000000),
                                               jnp.int32(1 << 30)))

    def sec_scan(sec, cnt):
        def g_body(g, cnt):
            rvec = ids_v[pl.ds(g * 16, 16)]
            pos = sec * 2048 + g * 16 + lanes
            m_a = (rvec >= v0) & (rvec < v0 + VS)
            m_b = (rvec >= lo2) & (rvec < hi2)
            rloc = jnp.where(m_b, rvec - lo2 + NWIN * CW, rvec - v0)
            packed = (rloc << 14) | pos
            m = m_a | m_b
            plsc.store_compressed(mrp_v.at[pl.ds(cnt, 16)], packed, mask=m)
            return cnt + plsc.all_reduce_population_count(m)[0]

        pltpu.sync_copy(idx_hbm.at[pl.ds(sec * 2048, 2048)], ids_v)
        return lax.fori_loop(0, 128, g_body, cnt)

    cnt = lax.fori_loop(0, 8, sec_scan, jnp.int32(0))

    def fire(win, buf):
        pltpu.async_copy(
            tT_hbm.at[:, pl.ds(v0 + win * CW, CW)], buf, semw)

    def flush(sr):
        def pad(g, _):
            pv = posw_v[0, pl.ds(g * 16, 16)]
            gl = g * 16 + lanes
            posw_v[0, pl.ds(g * 16, 16)] = jnp.where(gl < sr, pv, B + gl)
            return 0

        lax.fori_loop(0, 8, pad, 0)
        pltpu.async_copy(stage_v, out128_hbm.at[posw_v.at[0]], sems).wait()

    def window_pass(buf, wbase, sr0):
        def k_body(k, sr):
            pk = mrp_v[pl.ds(k * 16, 16)]
            valid = (k * 16 + lanes) < cnt
            rloc = pk >> 14
            m2 = valid & (rloc >= wbase) & (rloc < wbase + CW)
            mi = m2.astype(jnp.int32)
            for l in range(16):
                hit = mi[l] != 0

                @pl.when(hit)
                def _():
                    rr = rloc[l] - wbase
                    p = pk[l] & 16383
                    isp = jnp.full((16,), rr, dtype=jnp.int32)
                    for q in range(4):
                        vals = plsc.load_gather(buf, [lanes + q * 16, isp])
                        stage_v[sr, pl.ds(q * 16, 16)] = vals
                    g = sr // 16
                    pv = posw_v[0, pl.ds(g * 16, 16)]
                    posw_v[0, pl.ds(g * 16, 16)] = jnp.where(
                        lanes == sr % 16, p, pv)

                sr_n = jnp.where(hit, sr + 1, sr)

                @pl.when(sr_n == NSTAGE)
                def _():
                    flush(jnp.int32(NSTAGE))

                sr = jnp.where(sr_n == NSTAGE, 0, sr_n)
            return sr

        return lax.fori_loop(0, (cnt + 15) // 16, k_body, sr0)

    fire(jnp.int32(0), buf0_v)

    def win_body(w, sr):
        @pl.when(w < NWIN - 1)
        def _():
            b_n = (w + 1) % 2

            @pl.when(b_n == 0)
            def _():
                fire(w + 1, buf0_v)

            @pl.when(b_n == 1)
            def _():
                fire(w + 1, buf1_v)

        pltpu.make_async_copy(
            tT_hbm.at[:, pl.ds(0, CW)], buf0_v, semw).wait()
        b = w % 2

        def pass0(sr):
            return window_pass(buf0_v, w * CW, sr)

        def pass1(sr):
            return window_pass(buf1_v, w * CW, sr)

        return lax.cond(b == 0, pass0, pass1, sr)

    sr = lax.fori_loop(0, NWIN, win_body, jnp.int32(0))

    @pl.when(is30)
    def _():
        pltpu.async_copy(tT_hbm.at[:, pl.ds(LO30, CW)], buf0_v, semw).wait()

    @pl.when(is31)
    def _():
        pltpu.sync_copy(tail_hbm, tail_v)

    def no_pass(sr):
        return sr

    sr = lax.cond(is30, lambda s: window_pass(buf0_v, NWIN * CW, s),
                  no_pass, sr)
    sr = lax.cond(is31, lambda s: window_pass(tail_v, NWIN * CW, s),
                  no_pass, sr)

    flush(sr)


@functools.partial(
    pl.kernel,
    mesh=_mesh,
    out_type=jax.ShapeDtypeStruct((B, D), jnp.float32),
    scratch_types=[
        pltpu.VMEM((B_PER_W, 128), jnp.float32),
        pltpu.VMEM((B_PER_W, D), jnp.float32),
        pltpu.SemaphoreType.DMA,
    ],
)
def _strip(out128_hbm, out_hbm, sb_v, ob_v, sem):
    wid = lax.axis_index("s") * NC + lax.axis_index("c")
    base = wid * B_PER_W
    pltpu.sync_copy(out128_hbm.at[pl.ds(base, B_PER_W), :], sb_v)

    def row_copy(i, _):
        for q in range(D // 16):
            ob_v[i, pl.ds(q * 16, 16)] = sb_v[i, pl.ds(q * 16, 16)]
        return 0

    lax.fori_loop(0, B_PER_W, row_copy, 0)
    pltpu.sync_copy(ob_v, out_hbm.at[pl.ds(base, B_PER_W)])


def kernel(wl_ids, table):
    ids = wl_ids.astype(jnp.int32)
    t_t = table.T
    tail = jnp.pad(lax.slice(t_t, (0, LO31), (D, 1000000)),
                   ((0, 0), (0, 64)))
    out128 = _scan_extract(ids, t_t, tail)
    return _strip(out128)


# speedup vs baseline: 4.3097x; 4.3097x over previous
import functools

import jax
import jax.numpy as jnp
from jax import lax
from jax.experimental import pallas as pl
from jax.experimental.pallas import tpu as pltpu
from jax.experimental.pallas import tpu_sc as plsc

NC = 2
NS = 16
NW = NC * NS
B = 16384
D = 64
B_PER_W = B // NW
VS = 244 * 128
CW = 512
NWIN = VS // CW
LO30 = NW * VS
LO31 = LO30 + CW
NSTAGE = 128
OUT128_ROWS = B + NSTAGE

_mesh = plsc.VectorSubcoreMesh(core_axis_name="c", subcore_axis_name="s")


@functools.partial(
    pl.kernel,
    mesh=_mesh,
    out_type=jax.ShapeDtypeStruct((OUT128_ROWS, 128), jnp.float32),
    compiler_params=pltpu.CompilerParams(needs_layout_passes=False),
    scratch_types=[
        pltpu.VMEM((2048,), jnp.int32),
        pltpu.VMEM((B,), jnp.int32),
        pltpu.VMEM((D, CW), jnp.float32),
        pltpu.VMEM((D, CW), jnp.float32),
        pltpu.VMEM((NSTAGE, 128), jnp.float32),
        pltpu.VMEM((1, 128), jnp.int32),
        pltpu.VMEM((D, 128), jnp.float32),
        pltpu.VMEM((16,), jnp.int32),
        pltpu.SemaphoreType.DMA,
        pltpu.SemaphoreType.DMA,
    ],
)
def _scan_extract(idx_hbm, tT_hbm, tail_hbm, out128_hbm,
                  ids_v, mrp_v, buf0_v, buf1_v, stage_v, posw_v, tail_v,
                  hit_v, semw, sems):
    wid = lax.axis_index("s") * NC + lax.axis_index("c")
    v0 = wid * VS
    lanes = lax.iota(jnp.int32, 16)
    is30 = wid == 30
    is31 = wid == 31
    lo2 = jnp.where(is30, LO30, jnp.where(is31, LO31, jnp.int32(1 << 30)))
    hi2 = jnp.where(is30, LO30 + CW, jnp.where(is31, jnp.int32(1000000),
                                               jnp.int32(1 << 30)))

    def sec_scan(sec, cnt):
        def g_body(g, cnt):
            rvec = ids_v[pl.ds(g * 16, 16)]
            pos = sec * 2048 + g * 16 + lanes
            m_a = (rvec >= v0) & (rvec < v0 + VS)
            m_b = (rvec >= lo2) & (rvec < hi2)
            rloc = jnp.where(m_b, rvec - lo2 + NWIN * CW, rvec - v0)
            packed = (rloc << 14) | pos
            m = m_a | m_b
            plsc.store_compressed(mrp_v.at[pl.ds(cnt, 16)], packed, mask=m)
            return cnt + plsc.all_reduce_population_count(m)[0]

        pltpu.sync_copy(idx_hbm.at[pl.ds(sec * 2048, 2048)], ids_v)
        return lax.fori_loop(0, 128, g_body, cnt)

    cnt = lax.fori_loop(0, 8, sec_scan, jnp.int32(0))

    def fire(win, buf):
        pltpu.async_copy(
            tT_hbm.at[:, pl.ds(v0 + win * CW, CW)], buf, semw)

    def flush(sr):
        def pad(g, _):
            pv = posw_v[0, pl.ds(g * 16, 16)]
            gl = g * 16 + lanes
            posw_v[0, pl.ds(g * 16, 16)] = jnp.where(gl < sr, pv, B + gl)
            return 0

        lax.fori_loop(0, 8, pad, 0)
        pltpu.async_copy(stage_v, out128_hbm.at[posw_v.at[0]], sems).wait()

    def window_pass(buf, wbase, sr0):
        def k_body(k, sr):
            pk = mrp_v[pl.ds(k * 16, 16)]
            valid = (k * 16 + lanes) < cnt
            rloc = pk >> 14
            m2 = valid & (rloc >= wbase) & (rloc < wbase + CW)
            nh = plsc.all_reduce_population_count(m2)[0]

            def hitwork(sr):
                plsc.store_compressed(hit_v.at[pl.ds(0, 16)], pk, mask=m2)

                def h_body(i, sr):
                    pv16 = plsc.load_gather(
                        hit_v, [jnp.full((16,), i, dtype=jnp.int32)])
                    pki = pv16[0]
                    rr = (pki >> 14) - wbase
                    p = pki & 16383
                    isp = jnp.full((16,), rr, dtype=jnp.int32)
                    for q in range(4):
                        vals = plsc.load_gather(buf, [lanes + q * 16, isp])
                        stage_v[sr, pl.ds(q * 16, 16)] = vals
                    g = sr // 16
                    pv = posw_v[0, pl.ds(g * 16, 16)]
                    posw_v[0, pl.ds(g * 16, 16)] = jnp.where(
                        lanes == sr % 16, p, pv)
                    sr_n = sr + 1

                    @pl.when(sr_n == NSTAGE)
                    def _():
                        flush(jnp.int32(NSTAGE))

                    return jnp.where(sr_n == NSTAGE, 0, sr_n)

                return lax.fori_loop(0, nh, h_body, sr)

            return lax.cond(nh > 0, hitwork, lambda s: s, sr)

        return lax.fori_loop(0, (cnt + 15) // 16, k_body, sr0)

    fire(jnp.int32(0), buf0_v)

    def win_body(w, sr):
        @pl.when(w < NWIN - 1)
        def _():
            b_n = (w + 1) % 2

            @pl.when(b_n == 0)
            def _():
                fire(w + 1, buf0_v)

            @pl.when(b_n == 1)
            def _():
                fire(w + 1, buf1_v)

        pltpu.make_async_copy(
            tT_hbm.at[:, pl.ds(0, CW)], buf0_v, semw).wait()
        b = w % 2

        def pass0(sr):
            return window_pass(buf0_v, w * CW, sr)

        def pass1(sr):
            return window_pass(buf1_v, w * CW, sr)

        return lax.cond(b == 0, pass0, pass1, sr)

    sr = lax.fori_loop(0, NWIN, win_body, jnp.int32(0))

    @pl.when(is30)
    def _():
        pltpu.async_copy(tT_hbm.at[:, pl.ds(LO30, CW)], buf0_v, semw).wait()

    @pl.when(is31)
    def _():
        pltpu.sync_copy(tail_hbm, tail_v)

    def no_pass(sr):
        return sr

    sr = lax.cond(is30, lambda s: window_pass(buf0_v, NWIN * CW, s),
                  no_pass, sr)
    sr = lax.cond(is31, lambda s: window_pass(tail_v, NWIN * CW, s),
                  no_pass, sr)

    flush(sr)


@functools.partial(
    pl.kernel,
    mesh=_mesh,
    out_type=jax.ShapeDtypeStruct((B, D), jnp.float32),
    scratch_types=[
        pltpu.VMEM((B_PER_W, 128), jnp.float32),
        pltpu.VMEM((B_PER_W, D), jnp.float32),
        pltpu.SemaphoreType.DMA,
    ],
)
def _strip(out128_hbm, out_hbm, sb_v, ob_v, sem):
    wid = lax.axis_index("s") * NC + lax.axis_index("c")
    base = wid * B_PER_W
    pltpu.sync_copy(out128_hbm.at[pl.ds(base, B_PER_W), :], sb_v)

    def row_copy(i, _):
        for q in range(D // 16):
            ob_v[i, pl.ds(q * 16, 16)] = sb_v[i, pl.ds(q * 16, 16)]
        return 0

    lax.fori_loop(0, B_PER_W, row_copy, 0)
    pltpu.sync_copy(ob_v, out_hbm.at[pl.ds(base, B_PER_W)])


def kernel(wl_ids, table):
    ids = wl_ids.astype(jnp.int32)
    t_t = table.T
    tail = jnp.pad(lax.slice(t_t, (0, LO31), (D, 1000000)),
                   ((0, 0), (0, 64)))
    out128 = _scan_extract(ids, t_t, tail)
    return _strip(out128)


# speedup vs baseline: 4.4129x; 1.0240x over previous
import functools

import jax
import jax.numpy as jnp
from jax import lax
from jax.experimental import pallas as pl
from jax.experimental.pallas import tpu as pltpu
from jax.experimental.pallas import tpu_sc as plsc

NC = 2
NS = 16
NW = NC * NS
B = 16384
D = 64
B_PER_W = B // NW
VS = 244 * 128
CW = 512
NWIN = VS // CW
LO30 = NW * VS
LO31 = LO30 + CW
NSTAGE = 128
OUT128_ROWS = B + NSTAGE

_mesh = plsc.VectorSubcoreMesh(core_axis_name="c", subcore_axis_name="s")


@functools.partial(
    pl.kernel,
    mesh=_mesh,
    out_type=jax.ShapeDtypeStruct((OUT128_ROWS, 128), jnp.float32),
    compiler_params=pltpu.CompilerParams(needs_layout_passes=False),
    scratch_types=[
        pltpu.VMEM((2048,), jnp.int32),
        pltpu.VMEM((B + 64,), jnp.int32),
        pltpu.VMEM((D, CW), jnp.float32),
        pltpu.VMEM((D, CW), jnp.float32),
        pltpu.VMEM((NSTAGE, 128), jnp.float32),
        pltpu.VMEM((1, 128), jnp.int32),
        pltpu.VMEM((D, 128), jnp.float32),
        pltpu.VMEM((16,), jnp.int32),
        pltpu.SemaphoreType.DMA,
        pltpu.SemaphoreType.DMA,
    ],
)
def _scan_extract(idx_hbm, tT_hbm, tail_hbm, out128_hbm,
                  ids_v, mrp_v, buf0_v, buf1_v, stage_v, posw_v, tail_v,
                  hit_v, semw, sems):
    wid = lax.axis_index("s") * NC + lax.axis_index("c")
    v0 = wid * VS
    lanes = lax.iota(jnp.int32, 16)
    is30 = wid == 30
    is31 = wid == 31
    lo2 = jnp.where(is30, LO30, jnp.where(is31, LO31, jnp.int32(1 << 30)))
    hi2 = jnp.where(is30, LO30 + CW, jnp.where(is31, jnp.int32(1000000),
                                               jnp.int32(1 << 30)))

    def sec_scan(sec, cnt):
        def g_body(g, cnt):
            rvec = ids_v[pl.ds(g * 16, 16)]
            pos = sec * 2048 + g * 16 + lanes
            m_a = (rvec >= v0) & (rvec < v0 + VS)
            m_b = (rvec >= lo2) & (rvec < hi2)
            rloc = jnp.where(m_b, rvec - lo2 + NWIN * CW, rvec - v0)
            packed = (rloc << 14) | pos
            m = m_a | m_b
            plsc.store_compressed(mrp_v.at[pl.ds(cnt, 16)], packed, mask=m)
            return cnt + plsc.all_reduce_population_count(m)[0]

        pltpu.sync_copy(idx_hbm.at[pl.ds(sec * 2048, 2048)], ids_v)
        return lax.fori_loop(0, 128, g_body, cnt)

    cnt = lax.fori_loop(0, 8, sec_scan, jnp.int32(0))

    def fire(win, buf):
        pltpu.async_copy(
            tT_hbm.at[:, pl.ds(v0 + win * CW, CW)], buf, semw)

    def flush(sr):
        def pad(g, _):
            pv = posw_v[0, pl.ds(g * 16, 16)]
            gl = g * 16 + lanes
            posw_v[0, pl.ds(g * 16, 16)] = jnp.where(gl < sr, pv, B + gl)
            return 0

        lax.fori_loop(0, 8, pad, 0)
        pltpu.async_copy(stage_v, out128_hbm.at[posw_v.at[0]], sems).wait()

    def window_pass(buf, wbase, sr0):
        def extract16(pk, m2, sr):
            nh = plsc.all_reduce_population_count(m2)[0]

            def hitwork(sr):
                plsc.store_compressed(hit_v.at[pl.ds(0, 16)], pk, mask=m2)

                def h_body(i, sr):
                    pv16 = plsc.load_gather(
                        hit_v, [jnp.full((16,), i, dtype=jnp.int32)])
                    pki = pv16[0]
                    rr = (pki >> 14) - wbase
                    p = pki & 16383
                    isp = jnp.full((16,), rr, dtype=jnp.int32)
                    for q in range(4):
                        vals = plsc.load_gather(buf, [lanes + q * 16, isp])
                        stage_v[sr, pl.ds(q * 16, 16)] = vals
                    g = sr // 16
                    pv = posw_v[0, pl.ds(g * 16, 16)]
                    posw_v[0, pl.ds(g * 16, 16)] = jnp.where(
                        lanes == sr % 16, p, pv)
                    sr_n = sr + 1

                    @pl.when(sr_n == NSTAGE)
                    def _():
                        flush(jnp.int32(NSTAGE))

                    return jnp.where(sr_n == NSTAGE, 0, sr_n)

                return lax.fori_loop(0, nh, h_body, sr)

            return lax.cond(nh > 0, hitwork, lambda s: s, sr)

        def k_body(kg, sr):
            pks = [mrp_v[pl.ds((kg * 4 + t) * 16, 16)] for t in range(4)]
            ms = []
            m_any = None
            for t in range(4):
                valid = ((kg * 4 + t) * 16 + lanes) < cnt
                rloc = pks[t] >> 14
                m2 = valid & (rloc >= wbase) & (rloc < wbase + CW)
                ms.append(m2)
                m_any = m2 if m_any is None else (m_any | m2)
            na = plsc.all_reduce_population_count(m_any)[0]

            def groupwork(sr):
                for t in range(4):
                    sr = extract16(pks[t], ms[t], sr)
                return sr

            return lax.cond(na > 0, groupwork, lambda s: s, sr)

        return lax.fori_loop(0, (cnt + 63) // 64, k_body, sr0)

    fire(jnp.int32(0), buf0_v)

    def win_body(w, sr):
        @pl.when(w < NWIN - 1)
        def _():
            b_n = (w + 1) % 2

            @pl.when(b_n == 0)
            def _():
                fire(w + 1, buf0_v)

            @pl.when(b_n == 1)
            def _():
                fire(w + 1, buf1_v)

        pltpu.make_async_copy(
            tT_hbm.at[:, pl.ds(0, CW)], buf0_v, semw).wait()
        b = w % 2

        def pass0(sr):
            return window_pass(buf0_v, w * CW, sr)

        def pass1(sr):
            return window_pass(buf1_v, w * CW, sr)

        return lax.cond(b == 0, pass0, pass1, sr)

    sr = lax.fori_loop(0, NWIN, win_body, jnp.int32(0))

    @pl.when(is30)
    def _():
        pltpu.async_copy(tT_hbm.at[:, pl.ds(LO30, CW)], buf0_v, semw).wait()

    @pl.when(is31)
    def _():
        pltpu.sync_copy(tail_hbm, tail_v)

    def no_pass(sr):
        return sr

    sr = lax.cond(is30, lambda s: window_pass(buf0_v, NWIN * CW, s),
                  no_pass, sr)
    sr = lax.cond(is31, lambda s: window_pass(tail_v, NWIN * CW, s),
                  no_pass, sr)

    flush(sr)


@functools.partial(
    pl.kernel,
    mesh=_mesh,
    out_type=jax.ShapeDtypeStruct((B, D), jnp.float32),
    scratch_types=[
        pltpu.VMEM((B_PER_W, 128), jnp.float32),
        pltpu.VMEM((B_PER_W, D), jnp.float32),
        pltpu.SemaphoreType.DMA,
    ],
)
def _strip(out128_hbm, out_hbm, sb_v, ob_v, sem):
    wid = lax.axis_index("s") * NC + lax.axis_index("c")
    base = wid * B_PER_W
    pltpu.sync_copy(out128_hbm.at[pl.ds(base, B_PER_W), :], sb_v)

    def row_copy(i, _):
        for q in range(D // 16):
            ob_v[i, pl.ds(q * 16, 16)] = sb_v[i, pl.ds(q * 16, 16)]
        return 0

    lax.fori_loop(0, B_PER_W, row_copy, 0)
    pltpu.sync_copy(ob_v, out_hbm.at[pl.ds(base, B_PER_W)])


def kernel(wl_ids, table):
    ids = wl_ids.astype(jnp.int32)
    t_t = table.T
    tail = jnp.pad(lax.slice(t_t, (0, LO31), (D, 1000000)),
                   ((0, 0), (0, 64)))
    out128 = _scan_extract(ids, t_t, tail)
    return _strip(out128)


# speedup vs baseline: 4.4375x; 1.0056x over previous
import functools

import jax
import jax.numpy as jnp
from jax import lax
from jax.experimental import pallas as pl
from jax.experimental.pallas import tpu as pltpu
from jax.experimental.pallas import tpu_sc as plsc

NC = 2
NS = 16
NW = NC * NS
B = 16384
D = 64
B_PER_W = B // NW
VS = 244 * 128
CW = 512
NWIN = VS // CW
LO30 = NW * VS
LO31 = LO30 + CW
NSTAGE = 128
OUT128_ROWS = B + NSTAGE

_mesh = plsc.VectorSubcoreMesh(core_axis_name="c", subcore_axis_name="s")


@functools.partial(
    pl.kernel,
    mesh=_mesh,
    out_type=jax.ShapeDtypeStruct((OUT128_ROWS, 128), jnp.float32),
    compiler_params=pltpu.CompilerParams(needs_layout_passes=False),
    scratch_types=[
        pltpu.VMEM((2048,), jnp.int32),
        pltpu.VMEM((B + 64,), jnp.int32),
        pltpu.VMEM((D, CW), jnp.float32),
        pltpu.VMEM((D, CW), jnp.float32),
        pltpu.VMEM((NSTAGE, 128), jnp.float32),
        pltpu.VMEM((1, 128), jnp.int32),
        pltpu.VMEM((D, 128), jnp.float32),
        pltpu.VMEM((16,), jnp.int32),
        pltpu.SemaphoreType.DMA,
        pltpu.SemaphoreType.DMA,
    ],
)
def _scan_extract(idx_hbm, tT_hbm, tail_hbm, out128_hbm,
                  ids_v, mrp_v, buf0_v, buf1_v, stage_v, posw_v, tail_v,
                  hit_v, semw, sems):
    wid = lax.axis_index("s") * NC + lax.axis_index("c")
    v0 = wid * VS
    lanes = lax.iota(jnp.int32, 16)
    is30 = wid == 30
    is31 = wid == 31
    lo2 = jnp.where(is30, LO30, jnp.where(is31, LO31, jnp.int32(1 << 30)))
    hi2 = jnp.where(is30, LO30 + CW, jnp.where(is31, jnp.int32(1000000),
                                               jnp.int32(1 << 30)))

    def sec_scan_plain(sec, cnt):
        def g_body(g, cnt):
            rvec = ids_v[pl.ds(g * 16, 16)]
            pos = sec * 2048 + g * 16 + lanes
            rloc = rvec - v0
            m = (rloc >= 0) & (rloc < VS)
            packed = (rloc << 14) | pos
            plsc.store_compressed(mrp_v.at[pl.ds(cnt, 16)], packed, mask=m)
            return cnt + plsc.all_reduce_population_count(m)[0]

        pltpu.sync_copy(idx_hbm.at[pl.ds(sec * 2048, 2048)], ids_v)
        return lax.fori_loop(0, 128, g_body, cnt)

    def sec_scan_tail(sec, cnt):
        def g_body(g, cnt):
            rvec = ids_v[pl.ds(g * 16, 16)]
            pos = sec * 2048 + g * 16 + lanes
            m_a = (rvec >= v0) & (rvec < v0 + VS)
            m_b = (rvec >= lo2) & (rvec < hi2)
            rloc = jnp.where(m_b, rvec - lo2 + NWIN * CW, rvec - v0)
            packed = (rloc << 14) | pos
            m = m_a | m_b
            plsc.store_compressed(mrp_v.at[pl.ds(cnt, 16)], packed, mask=m)
            return cnt + plsc.all_reduce_population_count(m)[0]

        pltpu.sync_copy(idx_hbm.at[pl.ds(sec * 2048, 2048)], ids_v)
        return lax.fori_loop(0, 128, g_body, cnt)

    cnt = lax.cond(
        is30 | is31,
        lambda: lax.fori_loop(0, 8, sec_scan_tail, jnp.int32(0)),
        lambda: lax.fori_loop(0, 8, sec_scan_plain, jnp.int32(0)),
    )

    def fire(win, buf):
        pltpu.async_copy(
            tT_hbm.at[:, pl.ds(v0 + win * CW, CW)], buf, semw)

    def flush(sr):
        def pad(g, _):
            pv = posw_v[0, pl.ds(g * 16, 16)]
            gl = g * 16 + lanes
            posw_v[0, pl.ds(g * 16, 16)] = jnp.where(gl < sr, pv, B + gl)
            return 0

        lax.fori_loop(0, 8, pad, 0)
        pltpu.async_copy(stage_v, out128_hbm.at[posw_v.at[0]], sems).wait()

    def window_pass(buf, wbase, sr0):
        def extract16(pk, m2, sr):
            nh = plsc.all_reduce_population_count(m2)[0]

            def hitwork(sr):
                plsc.store_compressed(hit_v.at[pl.ds(0, 16)], pk, mask=m2)

                def h_body(i, sr):
                    pv16 = plsc.load_gather(
                        hit_v, [jnp.full((16,), i, dtype=jnp.int32)])
                    pki = pv16[0]
                    rr = (pki >> 14) - wbase
                    p = pki & 16383
                    isp = jnp.full((16,), rr, dtype=jnp.int32)
                    for q in range(4):
                        vals = plsc.load_gather(buf, [lanes + q * 16, isp])
                        stage_v[sr, pl.ds(q * 16, 16)] = vals
                    g = sr // 16
                    pv = posw_v[0, pl.ds(g * 16, 16)]
                    posw_v[0, pl.ds(g * 16, 16)] = jnp.where(
                        lanes == sr % 16, p, pv)
                    sr_n = sr + 1

                    @pl.when(sr_n == NSTAGE)
                    def _():
                        flush(jnp.int32(NSTAGE))

                    return jnp.where(sr_n == NSTAGE, 0, sr_n)

                return lax.fori_loop(0, nh, h_body, sr)

            return lax.cond(nh > 0, hitwork, lambda s: s, sr)

        def k_body(kg, sr):
            pks = [mrp_v[pl.ds((kg * 4 + t) * 16, 16)] for t in range(4)]
            ms = []
            m_any = None
            for t in range(4):
                valid = ((kg * 4 + t) * 16 + lanes) < cnt
                rloc = pks[t] >> 14
                m2 = valid & (rloc >= wbase) & (rloc < wbase + CW)
                ms.append(m2)
                m_any = m2 if m_any is None else (m_any | m2)
            na = plsc.all_reduce_population_count(m_any)[0]

            def groupwork(sr):
                for t in range(4):
                    sr = extract16(pks[t], ms[t], sr)
                return sr

            return lax.cond(na > 0, groupwork, lambda s: s, sr)

        return lax.fori_loop(0, (cnt + 63) // 64, k_body, sr0)

    fire(jnp.int32(0), buf0_v)

    def win_body(w, sr):
        @pl.when(w < NWIN - 1)
        def _():
            b_n = (w + 1) % 2

            @pl.when(b_n == 0)
            def _():
                fire(w + 1, buf0_v)

            @pl.when(b_n == 1)
            def _():
                fire(w + 1, buf1_v)

        pltpu.make_async_copy(
            tT_hbm.at[:, pl.ds(0, CW)], buf0_v, semw).wait()
        b = w % 2

        def pass0(sr):
            return window_pass(buf0_v, w * CW, sr)

        def pass1(sr):
            return window_pass(buf1_v, w * CW, sr)

        return lax.cond(b == 0, pass0, pass1, sr)

    sr = lax.fori_loop(0, NWIN, win_body, jnp.int32(0))

    @pl.when(is30)
    def _():
        pltpu.async_copy(tT_hbm.at[:, pl.ds(LO30, CW)], buf0_v, semw).wait()

    @pl.when(is31)
    def _():
        pltpu.sync_copy(tail_hbm, tail_v)

    def no_pass(sr):
        return sr

    sr = lax.cond(is30, lambda s: window_pass(buf0_v, NWIN * CW, s),
                  no_pass, sr)
    sr = lax.cond(is31, lambda s: window_pass(tail_v, NWIN * CW, s),
                  no_pass, sr)

    flush(sr)


@functools.partial(
    pl.kernel,
    mesh=_mesh,
    out_type=jax.ShapeDtypeStruct((B, D), jnp.float32),
    scratch_types=[
        pltpu.VMEM((B_PER_W, 128), jnp.float32),
        pltpu.VMEM((B_PER_W, D), jnp.float32),
        pltpu.SemaphoreType.DMA,
    ],
)
def _strip(out128_hbm, out_hbm, sb_v, ob_v, sem):
    wid = lax.axis_index("s") * NC + lax.axis_index("c")
    base = wid * B_PER_W
    pltpu.sync_copy(out128_hbm.at[pl.ds(base, B_PER_W), :], sb_v)

    def row_copy(i, _):
        for q in range(D // 16):
            ob_v[i, pl.ds(q * 16, 16)] = sb_v[i, pl.ds(q * 16, 16)]
        return 0

    lax.fori_loop(0, B_PER_W, row_copy, 0)
    pltpu.sync_copy(ob_v, out_hbm.at[pl.ds(base, B_PER_W)])


def kernel(wl_ids, table):
    ids = wl_ids.astype(jnp.int32)
    t_t = table.T
    tail = jnp.pad(lax.slice(t_t, (0, LO31), (D, 1000000)),
                   ((0, 0), (0, 64)))
    out128 = _scan_extract(ids, t_t, tail)
    return _strip(out128)
